# baseline (speedup 1.0000x reference)
import functools

import jax
import jax.numpy as jnp
from jax import lax
from jax.experimental import pallas as pl
from jax.experimental.pallas import tpu as pltpu
from jax.experimental.pallas import tpu_sc as plsc

B = 4096
NBALL = 131072
E = 131072
H = 128
L = 3

_NC = 2
_NS = 16
_NW = _NC * _NS
_K = 128
_F32 = jnp.float32


def _row_seg_accum(table, src3, dst3, n_items, n_segs, mode,
                   ids_flat=None, den_vals=None, item_off=0):
    per_w = n_items // _NW
    n_chunks = per_w // _K
    rows_per_sub = n_segs // _NS
    seg_rows = n_segs // _K
    with_den = den_vals is not None
    mesh = plsc.VectorSubcoreMesh(core_axis_name="c", subcore_axis_name="s",
                                  num_cores=_NC, num_subcores=_NS)

    def body(*refs):
        if with_den:
            (table_ref, ids_ref, vals_ref,
             out_ref, den_ref,
             dst_v0, dst_v1, rows0, rows1, acc, sem0, sem1,
             ids_buf, vals2d, s2d, e2d, idx_vm, den_acc) = refs
            src_vm = dst_vm = None
        else:
            (table_ref, src_ref, dst_ref, out_ref,
             src_vm, dst_vm, rows0, rows1, acc, sem0, sem1) = refs
        c = lax.axis_index("c")
        s = lax.axis_index("s")
        wid = c * _NS + s

        def zero_row(i, carry):
            for j in range(H // 16):
                rows0[i, pl.ds(j * 16, 16)] = jnp.zeros((16,), _F32)
            return carry
        lax.fori_loop(0, _K, zero_row, 0)
        for t in range(rows_per_sub // _K):
            pltpu.sync_copy(rows0, acc.at[pl.ds(s * rows_per_sub + t * _K, _K)])
        if with_den:
            @pl.when(s == 0)
            def _():
                pltpu.sync_copy(rows0.at[pl.ds(0, seg_rows)], den_acc)
        plsc.subcore_barrier()

        if with_den:
            neg1 = jnp.full((16,), -1, jnp.int32)
            ids_buf[pl.ds(0, 16)] = neg1
            ids_buf[pl.ds(16 + per_w, 16)] = neg1
            off0 = pl.multiple_of(item_off + wid * per_w, _K)
            pltpu.sync_copy(ids_ref.at[pl.ds(off0, per_w)],
                            ids_buf.at[pl.ds(16, per_w)])
            pltpu.sync_copy(vals_ref.at[pl.ds(wid * (per_w // _K),
                                              per_w // _K)], vals2d)
        else:
            pltpu.sync_copy(src_ref.at[wid], src_vm)
            pltpu.sync_copy(dst_ref.at[wid], dst_vm)

        def fetch(i, buf, sem):
            if mode == "gather":
                return pltpu.async_copy(table_ref.at[src_vm.at[i]], buf, sem)
            off = pl.multiple_of(wid * per_w, _K) + i * _K
            return pltpu.async_copy(table_ref.at[pl.ds(off, _K)], buf, sem)

        def wait(i, buf, sem):
            if mode == "gather":
                pltpu.make_async_copy(table_ref.at[src_vm.at[i]], buf, sem).wait()
            else:
                off = pl.multiple_of(wid * per_w, _K) + i * _K
                pltpu.make_async_copy(table_ref.at[pl.ds(off, _K)], buf, sem).wait()

        def dst_idx(i, which):
            if not with_den:
                return dst_vm.at[i]
            buf = dst_v0 if which == 0 else dst_v1
            off = pl.multiple_of(item_off + wid * per_w, _K) + i * _K
            pltpu.sync_copy(ids_ref.at[pl.ds(off, _K)], buf)
            return buf

        fetch(0, rows0, sem0)

        def chunk2(k, carry):
            i0 = 2 * k
            i1 = 2 * k + 1
            wait(i0, rows0, sem0)
            fetch(i1, rows1, sem1)
            pltpu.sync_copy(rows0, acc.at[dst_idx(i0, 0)], add=True)
            wait(i1, rows1, sem1)

            @pl.when(i0 + 2 < n_chunks)
            def _():
                fetch(i0 + 2, rows0, sem0)
            pltpu.sync_copy(rows1, acc.at[dst_idx(i1, 1)], add=True)
            return carry
        lax.fori_loop(0, n_chunks // 2, chunk2, 0)

        if with_den:
            def zero2d(r, carry):
                for j in range(_K // 16):
                    s2d[r, pl.ds(j * 16, 16)] = jnp.zeros((16,), _F32)
                    e2d[r, pl.ds(j * 16, 16)] = jnp.zeros((16,), _F32)
                return carry
            lax.fori_loop(0, seg_rows, zero2d, 0)
            idx_vm[pl.ds(0, 16)] = lax.iota(jnp.int32, 16)
            idx_vm[pl.ds(16, 16)] = lax.iota(jnp.int32, 16) + 16

            def group(g, carry):
                base = g * 16
                ids = ids_buf[pl.ds(16 + base, 16)]
                prv = ids_buf[pl.ds(15 + base, 16)]
                nxt = ids_buf[pl.ds(17 + base, 16)]
                v = vals2d[g // 8, pl.ds((g % 8) * 16, 16)]
                pc = plsc.cumsum(v) + carry
                hi = lax.shift_right_arithmetic(ids, 7)
                lo = lax.bitwise_and(ids, 127)
                plsc.store_scatter(s2d, [hi, lo], pc - v, mask=ids != prv)
                plsc.store_scatter(e2d, [hi, lo], pc, mask=ids != nxt)
                return pc[15]
            lax.fori_loop(0, per_w // 16, group, jnp.float32(0.0))

            def diff(r, carry):
                for j in range(_K // 16):
                    sl = pl.ds(j * 16, 16)
                    e2d[r, sl] = e2d[r, sl] - s2d[r, sl]
                return carry
            lax.fori_loop(0, seg_rows, diff, 0)
            pltpu.sync_copy(e2d, den_acc.at[idx_vm], add=True)

        plsc.subcore_barrier()
        for t in range(rows_per_sub // _K):
            r0 = s * rows_per_sub + t * _K
            pltpu.sync_copy(acc.at[pl.ds(r0, _K)], out_ref.at[c, pl.ds(r0, _K)])
        if with_den:
            @pl.when(s == 0)
            def _():
                pltpu.sync_copy(den_acc, den_ref.at[c])

    out_type = jax.ShapeDtypeStruct((_NC, n_segs, H), _F32)
    if with_den:
        out_type = [out_type,
                    jax.ShapeDtypeStruct((_NC, seg_rows, _K), _F32)]
        scratch = [
            pltpu.VMEM((_K,), jnp.int32),
            pltpu.VMEM((_K,), jnp.int32),
            pltpu.VMEM((_K, H), _F32),
            pltpu.VMEM((_K, H), _F32),
            pltpu.VMEM_SHARED((n_segs, H), _F32),
            pltpu.SemaphoreType.DMA,
            pltpu.SemaphoreType.DMA,
            pltpu.VMEM((per_w + 32,), jnp.int32),
            pltpu.VMEM((per_w // _K, _K), _F32),
            pltpu.VMEM((seg_rows, _K), _F32),
            pltpu.VMEM((seg_rows, _K), _F32),
            pltpu.VMEM((seg_rows,), jnp.int32),
            pltpu.VMEM_SHARED((seg_rows, _K), _F32),
        ]
    else:
        scratch = [
            pltpu.VMEM((n_chunks, _K), jnp.int32),
            pltpu.VMEM((n_chunks, _K), jnp.int32),
            pltpu.VMEM((_K, H), _F32),
            pltpu.VMEM((_K, H), _F32),
            pltpu.VMEM_SHARED((n_segs, H), _F32),
            pltpu.SemaphoreType.DMA,
            pltpu.SemaphoreType.DMA,
        ]
    fn = pl.kernel(
        body,
        out_type=out_type,
        mesh=mesh,
        compiler_params=pltpu.CompilerParams(needs_layout_passes=False),
        scratch_types=scratch,
        name=f"row_seg_accum_{mode}" + (f"_den{item_off}" if with_den else ""),
    )
    if with_den:
        return fn(table, ids_flat, den_vals)
    return fn(table, src3, dst3)


def _scalar_seg_sum(ids2, vals2, n_items, n_segs):
    per_w = n_items // _NW
    n_groups = per_w // 16
    seg_rows = n_segs // _K
    mesh = plsc.VectorSubcoreMesh(core_axis_name="c", subcore_axis_name="s",
                                  num_cores=_NC, num_subcores=_NS)
    have_vals = vals2 is not None

    def body(*refs):
        if have_vals:
            (ids_ref, vals_ref, out_ref, ids_buf, vals_buf, s2d, e2d,
             idx_vm, zrow, acc) = refs
        else:
            (ids_ref, out_ref, ids_buf, vals_buf, s2d, e2d,
             idx_vm, zrow, acc) = refs
        c = lax.axis_index("c")
        s = lax.axis_index("s")
        wid = c * _NS + s

        neg1 = jnp.full((16,), -1, jnp.int32)
        ids_buf[pl.ds(0, 16)] = neg1
        ids_buf[pl.ds(16 + per_w, 16)] = neg1
        off = pl.multiple_of(wid * per_w, _K)
        pltpu.sync_copy(ids_ref.at[pl.ds(off, per_w)],
                        ids_buf.at[pl.ds(16, per_w)])
        if have_vals:
            pltpu.sync_copy(vals_ref.at[pl.ds(off, per_w)], vals_buf)

        def zero2d(r, carry):
            for j in range(_K // 16):
                s2d[r, pl.ds(j * 16, 16)] = jnp.zeros((16,), _F32)
                e2d[r, pl.ds(j * 16, 16)] = jnp.zeros((16,), _F32)
            return carry
        lax.fori_loop(0, seg_rows, zero2d, 0)
        for j in range(_K // 16):
            zrow[pl.ds(j * 16, 16)] = jnp.zeros((16,), _F32)
        idx_vm[pl.ds(0, 16)] = lax.iota(jnp.int32, 16)
        idx_vm[pl.ds(16, 16)] = lax.iota(jnp.int32, 16) + 16

        @pl.when(s == 0)
        def _():
            for r in range(seg_rows):
                pltpu.sync_copy(zrow, acc.at[r])
        plsc.subcore_barrier()

        ones16 = jnp.ones((16,), _F32)

        def group(g, carry):
            base = g * 16
            ids = ids_buf[pl.ds(16 + base, 16)]
            prv = ids_buf[pl.ds(15 + base, 16)]
            nxt = ids_buf[pl.ds(17 + base, 16)]
            v = vals_buf[pl.ds(base, 16)] if have_vals else ones16
            pc = plsc.cumsum(v) + carry
            hi = lax.shift_right_arithmetic(ids, 7)
            lo = lax.bitwise_and(ids, 127)
            plsc.store_scatter(s2d, [hi, lo], pc - v, mask=ids != prv)
            plsc.store_scatter(e2d, [hi, lo], pc, mask=ids != nxt)
            return pc[15]
        lax.fori_loop(0, n_groups, group, jnp.float32(0.0))

        def diff(r, carry):
            for j in range(_K // 16):
                sl = pl.ds(j * 16, 16)
                e2d[r, sl] = e2d[r, sl] - s2d[r, sl]
            return carry
        lax.fori_loop(0, seg_rows, diff, 0)
        pltpu.sync_copy(e2d, acc.at[idx_vm], add=True)
        plsc.subcore_barrier()

        @pl.when(s == 0)
        def _():
            pltpu.sync_copy(acc, out_ref.at[c])

    scratch = [
        pltpu.VMEM((per_w + 32,), jnp.int32),
        pltpu.VMEM((per_w,), _F32),
        pltpu.VMEM((seg_rows, _K), _F32),
        pltpu.VMEM((seg_rows, _K), _F32),
        pltpu.VMEM((seg_rows,), jnp.int32),
        pltpu.VMEM((_K,), _F32),
        pltpu.VMEM_SHARED((seg_rows, _K), _F32),
    ]
    fn = pl.kernel(
        body,
        out_type=jax.ShapeDtypeStruct((_NC, seg_rows, _K), _F32),
        mesh=mesh,
        compiler_params=pltpu.CompilerParams(needs_layout_passes=False),
        scratch_types=scratch,
        name="scalar_seg_sum" + ("_vals" if have_vals else "_ones"),
    )
    return fn(ids2, vals2) if have_vals else fn(ids2)


def _encode(xt, W, b, br):
    d, n = xt.shape

    def body(xt_ref, w_ref, b_ref, o_ref):
        o_ref[...] = jnp.maximum(
            lax.dot_general(xt_ref[...], w_ref[...],
                            (((0,), (0,)), ((), ())),
                            preferred_element_type=_F32)
            + b_ref[...], 0.0)

    return pl.pallas_call(
        body,
        grid=(n // br,),
        in_specs=[pl.BlockSpec((d, br), lambda i: (0, i)),
                  pl.BlockSpec((d, H), lambda i: (0, 0)),
                  pl.BlockSpec((1, H), lambda i: (0, 0))],
        out_specs=pl.BlockSpec((br, H), lambda i: (i, 0)),
        out_shape=jax.ShapeDtypeStruct((n, H), _F32),
    )(xt, W, b)


def _self_update(hb, W, b, br):
    n = hb.shape[0]

    def body(x_ref, w_ref, b_ref, o_ref):
        x = x_ref[...]
        o_ref[...] = x + jnp.maximum(
            jnp.dot(x, w_ref[...], preferred_element_type=_F32) + b_ref[...],
            0.0)

    return pl.pallas_call(
        body,
        grid=(n // br,),
        in_specs=[pl.BlockSpec((br, H), lambda i: (i, 0)),
                  pl.BlockSpec((H, H), lambda i: (0, 0)),
                  pl.BlockSpec((1, H), lambda i: (0, 0))],
        out_specs=pl.BlockSpec((br, H), lambda i: (i, 0)),
        out_shape=jax.ShapeDtypeStruct((n, H), _F32),
    )(hb, W, b)


def _upd2_score(hb, W2, b2, Wa1, ba1, Wa2, ba2, br, row0, nrows):
    n = nrows
    blk0 = row0 // br

    def body(hb_ref, w2u_ref, b2u_ref, w1_ref, b1_ref, w2_ref, b2_ref,
             y_ref, ex_ref):
        h2 = hb_ref[...]
        h3 = h2 + jnp.maximum(
            jnp.dot(h2, w2u_ref[...], preferred_element_type=_F32)
            + b2u_ref[...], 0.0)
        t = jnp.tanh(jnp.dot(h3, w1_ref[...], preferred_element_type=_F32)
                     + b1_ref[...])
        sc = jnp.dot(t, w2_ref[...], preferred_element_type=_F32) + b2_ref[...]
        ex = jnp.exp(sc)
        y_ref[...] = h3 * ex
        ex_ref[...] = jnp.reshape(ex, (br // _K, _K))

    return pl.pallas_call(
        body,
        grid=(n // br,),
        in_specs=[pl.BlockSpec((br, H), lambda i: (i + blk0, 0)),
                  pl.BlockSpec((H, H), lambda i: (0, 0)),
                  pl.BlockSpec((1, H), lambda i: (0, 0)),
                  pl.BlockSpec((H, H // 2), lambda i: (0, 0)),
                  pl.BlockSpec((1, H // 2), lambda i: (0, 0)),
                  pl.BlockSpec((H // 2, 1), lambda i: (0, 0)),
                  pl.BlockSpec((1, 1), lambda i: (0, 0))],
        out_specs=[pl.BlockSpec((br, H), lambda i: (i, 0)),
                   pl.BlockSpec((br // _K, _K), lambda i: (i, 0))],
        out_shape=[jax.ShapeDtypeStruct((n, H), _F32),
                   jax.ShapeDtypeStruct((n // _K, _K), _F32)],
    )(hb, W2, b2, Wa1, ba1, Wa2, ba2)


def _ln_blk(x, g, b):
    m = jnp.mean(x, axis=-1, keepdims=True)
    v = jnp.mean((x - m) ** 2, axis=-1, keepdims=True)
    return (x - m) / jnp.sqrt(v + 1e-5) * g + b


def _head(xq, aggs, degp, poolp, denp, Wq, bq, W_upd, b_upd, ln_g, ln_b,
          Wc, bc, ln_gc, ln_bc, Wbo1, bbo1, Wbo2, bbo2,
          Wwk1, bwk1, Wwk2, bwk2, br):
    dq = xq.shape[1]

    def body(xq_ref, a0_ref, a1_ref, a2_ref, dg_ref,
             ppa_ref, ppb_ref, dna_ref, dnb_ref,
             wq_ref, bq_ref, wu_ref, bu_ref, lg_ref, lb_ref,
             wc_ref, bc_ref, lgc_ref, lbc_ref,
             w1b_ref, b1b_ref, w2b_ref, b2b_ref,
             w1k_ref, b1k_ref, w2k_ref, b2k_ref,
             ob_ref, ow_ref):
        hq = jnp.maximum(
            jnp.dot(xq_ref[...], wq_ref[...], preferred_element_type=_F32)
            + bq_ref[...], 0.0)
        dg = dg_ref[...]
        deg = jnp.maximum(dg[0] + dg[1], 1.0)
        wu = wu_ref[...]
        bu = bu_ref[...]
        lg = lg_ref[...]
        lb = lb_ref[...]
        for l, a_ref in enumerate((a0_ref, a1_ref, a2_ref)):
            a = a_ref[...]
            agg = (a[0] + a[1]) / deg
            u = jnp.maximum(
                jnp.dot(agg, wu[l], preferred_element_type=_F32) + bu[l], 0.0)
            hq = _ln_blk(hq + u, lg[l], lb[l])
        ppa = ppa_ref[...]
        ppb = ppb_ref[...]
        dna = dna_ref[...]
        dnb = dnb_ref[...]
        den = dna[0] + dna[1] + dnb[0] + dnb[1]
        pooled = (ppa[0] + ppa[1] + ppb[0] + ppb[1]) / (den + 1e-16)
        wc = wc_ref[...]
        comb = (jnp.dot(hq, wc[0:H], preferred_element_type=_F32)
                + jnp.dot(pooled, wc[H:2 * H], preferred_element_type=_F32)
                + bc_ref[...])
        r = jax.nn.gelu(_ln_blk(comb, lgc_ref[...], lbc_ref[...]))
        hb1 = jnp.maximum(
            jnp.dot(r, w1b_ref[...], preferred_element_type=_F32)
            + b1b_ref[...], 0.0)
        ob_ref[...] = (jnp.dot(hb1, w2b_ref[...], preferred_element_type=_F32)
                       + b2b_ref[...])
        hk1 = jnp.maximum(
            jnp.dot(r, w1k_ref[...], preferred_element_type=_F32)
            + b1k_ref[...], 0.0)
        ow_ref[...] = (jnp.dot(hk1, w2k_ref[...], preferred_element_type=_F32)
                       + b2k_ref[...])

    full = lambda *shape: pl.BlockSpec(shape, lambda i: (0,) * len(shape))
    return pl.pallas_call(
        body,
        grid=(B // br,),
        in_specs=[
            pl.BlockSpec((br, dq), lambda i: (i, 0)),
            pl.BlockSpec((2, br, H), lambda i: (0, i, 0)),
            pl.BlockSpec((2, br, H), lambda i: (0, i, 0)),
            pl.BlockSpec((2, br, H), lambda i: (0, i, 0)),
            pl.BlockSpec((2, br, 1), lambda i: (0, i, 0)),
            pl.BlockSpec((2, br, H), lambda i: (0, i, 0)),
            pl.BlockSpec((2, br, H), lambda i: (0, i, 0)),
            pl.BlockSpec((2, br, 1), lambda i: (0, i, 0)),
            pl.BlockSpec((2, br, 1), lambda i: (0, i, 0)),
            full(dq, H), full(1, H),
            full(L, H, H), full(L, H), full(L, H), full(L, H),
            full(2 * H, H), full(1, H), full(1, H), full(1, H),
            full(H, H // 2), full(1, H // 2), full(H // 2, 1), full(1, 1),
            full(H, H // 2), full(1, H // 2), full(H // 2, 1), full(1, 1),
        ],
        out_specs=[pl.BlockSpec((br, 1), lambda i: (i, 0)),
                   pl.BlockSpec((br, 1), lambda i: (i, 0))],
        out_shape=[jax.ShapeDtypeStruct((B, 1), _F32),
                   jax.ShapeDtypeStruct((B, 1), _F32)],
    )(xq, aggs[0], aggs[1], aggs[2], degp, poolp[0], poolp[1],
      denp[0], denp[1], Wq, bq, W_upd, b_upd,
      ln_g, ln_b, Wc, bc, ln_gc, ln_bc, Wbo1, bbo1, Wbo2, bbo2,
      Wwk1, bwk1, Wwk2, bwk2)


def kernel(x_query, x_ball, edge_src, edge_dst, ball_batch,
           Wq, bq, Wb, bb, W_upd, b_upd, W_self, b_self, ln_g, ln_b,
           Wa1, ba1, Wa2, ba2, Wc, bc, ln_gc, ln_bc,
           Wbo1, bbo1, Wbo2, bbo2, Wwk1, bwk1, Wwk2, bwk2):
    n_chunks = E // _NW // _K

    src3 = edge_src.reshape(_NW, n_chunks, _K)
    dst3 = edge_dst.reshape(_NW, n_chunks, _K)

    hb0 = _encode(x_ball.T, Wb, bb.reshape(1, H), br=4096)
    degp = _scalar_seg_sum(edge_dst, None, n_items=E, n_segs=B)
    hb0c, _ = lax.optimization_barrier((hb0, degp))
    agg0 = _row_seg_accum(hb0c, src3, dst3, n_items=E, n_segs=B, mode="gather")
    hb1 = _self_update(hb0, W_self[0], b_self[0].reshape(1, H), br=4096)
    hb1c, _ = lax.optimization_barrier((hb1, agg0))
    agg1 = _row_seg_accum(hb1c, src3, dst3, n_items=E, n_segs=B, mode="gather")
    hb2 = _self_update(hb1, W_self[1], b_self[1].reshape(1, H), br=4096)
    hb2c, _ = lax.optimization_barrier((hb2, agg1))
    agg2 = _row_seg_accum(hb2c, src3, dst3, n_items=E, n_segs=B, mode="gather")
    aggs = [agg0, agg1, agg2]

    half = NBALL // 2
    sargs = (W_self[2], b_self[2].reshape(1, H), Wa1,
             ba1.reshape(1, H // 2), Wa2, ba2.reshape(1, 1))
    ya, exa = _upd2_score(hb2, *sargs, br=4096, row0=0, nrows=half)
    yb, exb = _upd2_score(hb2, *sargs, br=4096, row0=half, nrows=half)
    yac, _ = lax.optimization_barrier((ya, agg2))
    pool_a, den_a = _row_seg_accum(yac, None, None, n_items=half, n_segs=B,
                                   mode="linear", ids_flat=ball_batch,
                                   den_vals=exa, item_off=0)
    ybc, _ = lax.optimization_barrier((yb, pool_a))
    pool_b, den_b = _row_seg_accum(ybc, None, None, n_items=half, n_segs=B,
                                   mode="linear", ids_flat=ball_batch,
                                   den_vals=exb, item_off=half)
    poolp = (pool_a, pool_b)
    denp = (den_a.reshape(_NC, B, 1), den_b.reshape(_NC, B, 1))

    xq = jnp.pad(x_query, ((0, 0), (0, 1)))
    Wqp = jnp.pad(Wq, ((0, 1), (0, 0)))
    boundary, wicket = _head(
        xq, aggs, degp.reshape(_NC, B, 1), poolp, denp,
        Wqp, bq.reshape(1, H), W_upd, b_upd,
        ln_g, ln_b, Wc, bc.reshape(1, H), ln_gc.reshape(1, H),
        ln_bc.reshape(1, H), Wbo1, bbo1.reshape(1, H // 2), Wbo2,
        bbo2.reshape(1, 1), Wwk1, bwk1.reshape(1, H // 2), Wwk2,
        bwk2.reshape(1, 1), br=1024)
    return boundary, wicket


# speedup vs baseline: 1.0148x; 1.0148x over previous
import functools

import jax
import jax.numpy as jnp
from jax import lax
from jax.experimental import pallas as pl
from jax.experimental.pallas import tpu as pltpu
from jax.experimental.pallas import tpu_sc as plsc

B = 4096
NBALL = 131072
E = 131072
H = 128
L = 3

_NC = 2
_NS = 16
_NW = _NC * _NS
_K = 128
_F32 = jnp.float32


def _row_seg_accum(table, src3, dst3, n_items, n_segs, mode,
                   ids_flat=None, den_vals=None, item_off=0):
    per_w = n_items // _NW
    n_chunks = per_w // _K
    rows_per_sub = n_segs // _NS
    seg_rows = n_segs // _K
    with_den = den_vals is not None
    mesh = plsc.VectorSubcoreMesh(core_axis_name="c", subcore_axis_name="s",
                                  num_cores=_NC, num_subcores=_NS)

    def body(*refs):
        if with_den:
            (table_ref, ids_ref, vals_ref,
             out_ref, den_ref,
             dst_v0, dst_v1, rows0, rows1, acc, sem0, sem1,
             ids_buf, vals2d, s2d, e2d, idx_vm, den_acc) = refs
            src_vm = dst_vm = None
        else:
            (table_ref, src_ref, dst_ref, out_ref,
             src_vm, dst_vm, rows0, rows1, acc, sem0, sem1) = refs
        c = lax.axis_index("c")
        s = lax.axis_index("s")
        wid = c * _NS + s

        def zero_row(i, carry):
            for j in range(H // 16):
                rows0[i, pl.ds(j * 16, 16)] = jnp.zeros((16,), _F32)
            return carry
        lax.fori_loop(0, _K, zero_row, 0)
        for t in range(rows_per_sub // _K):
            pltpu.sync_copy(rows0, acc.at[pl.ds(s * rows_per_sub + t * _K, _K)])
        if with_den:
            @pl.when(s == 0)
            def _():
                pltpu.sync_copy(rows0.at[pl.ds(0, seg_rows)], den_acc)
        plsc.subcore_barrier()

        if with_den:
            neg1 = jnp.full((16,), -1, jnp.int32)
            ids_buf[pl.ds(0, 16)] = neg1
            ids_buf[pl.ds(16 + per_w, 16)] = neg1
            off0 = pl.multiple_of(item_off + wid * per_w, _K)
            pltpu.sync_copy(ids_ref.at[pl.ds(off0, per_w)],
                            ids_buf.at[pl.ds(16, per_w)])
            pltpu.sync_copy(vals_ref.at[pl.ds(wid * (per_w // _K),
                                              per_w // _K)], vals2d)
        else:
            pltpu.sync_copy(src_ref.at[wid], src_vm)
            pltpu.sync_copy(dst_ref.at[wid], dst_vm)

        def fetch(i, buf, sem):
            if mode == "gather":
                return pltpu.async_copy(table_ref.at[src_vm.at[i]], buf, sem)
            off = pl.multiple_of(wid * per_w, _K) + i * _K
            return pltpu.async_copy(table_ref.at[pl.ds(off, _K)], buf, sem)

        def wait(i, buf, sem):
            if mode == "gather":
                pltpu.make_async_copy(table_ref.at[src_vm.at[i]], buf, sem).wait()
            else:
                off = pl.multiple_of(wid * per_w, _K) + i * _K
                pltpu.make_async_copy(table_ref.at[pl.ds(off, _K)], buf, sem).wait()

        def dst_idx(i, which):
            if not with_den:
                return dst_vm.at[i]
            buf = dst_v0 if which == 0 else dst_v1
            off = pl.multiple_of(item_off + wid * per_w, _K) + i * _K
            pltpu.sync_copy(ids_ref.at[pl.ds(off, _K)], buf)
            return buf

        fetch(0, rows0, sem0)

        def chunk2(k, carry):
            i0 = 2 * k
            i1 = 2 * k + 1
            wait(i0, rows0, sem0)
            fetch(i1, rows1, sem1)
            pltpu.sync_copy(rows0, acc.at[dst_idx(i0, 0)], add=True)
            wait(i1, rows1, sem1)

            @pl.when(i0 + 2 < n_chunks)
            def _():
                fetch(i0 + 2, rows0, sem0)
            pltpu.sync_copy(rows1, acc.at[dst_idx(i1, 1)], add=True)
            return carry
        lax.fori_loop(0, n_chunks // 2, chunk2, 0)

        if with_den:
            def zero2d(r, carry):
                for j in range(_K // 16):
                    s2d[r, pl.ds(j * 16, 16)] = jnp.zeros((16,), _F32)
                    e2d[r, pl.ds(j * 16, 16)] = jnp.zeros((16,), _F32)
                return carry
            lax.fori_loop(0, seg_rows, zero2d, 0)
            idx_vm[pl.ds(0, 16)] = lax.iota(jnp.int32, 16)
            idx_vm[pl.ds(16, 16)] = lax.iota(jnp.int32, 16) + 16

            def group(g, carry):
                base = g * 16
                ids = ids_buf[pl.ds(16 + base, 16)]
                prv = ids_buf[pl.ds(15 + base, 16)]
                nxt = ids_buf[pl.ds(17 + base, 16)]
                v = vals2d[g // 8, pl.ds((g % 8) * 16, 16)]
                pc = plsc.cumsum(v) + carry
                hi = lax.shift_right_arithmetic(ids, 7)
                lo = lax.bitwise_and(ids, 127)
                plsc.store_scatter(s2d, [hi, lo], pc - v, mask=ids != prv)
                plsc.store_scatter(e2d, [hi, lo], pc, mask=ids != nxt)
                return pc[15]
            lax.fori_loop(0, per_w // 16, group, jnp.float32(0.0))

            def diff(r, carry):
                for j in range(_K // 16):
                    sl = pl.ds(j * 16, 16)
                    e2d[r, sl] = e2d[r, sl] - s2d[r, sl]
                return carry
            lax.fori_loop(0, seg_rows, diff, 0)
            pltpu.sync_copy(e2d, den_acc.at[idx_vm], add=True)

        plsc.subcore_barrier()
        for t in range(rows_per_sub // _K):
            r0 = s * rows_per_sub + t * _K
            pltpu.sync_copy(acc.at[pl.ds(r0, _K)], out_ref.at[c, pl.ds(r0, _K)])
        if with_den:
            @pl.when(s == 0)
            def _():
                pltpu.sync_copy(den_acc, den_ref.at[c])

    out_type = jax.ShapeDtypeStruct((_NC, n_segs, H), _F32)
    if with_den:
        out_type = [out_type,
                    jax.ShapeDtypeStruct((_NC, seg_rows, _K), _F32)]
        scratch = [
            pltpu.VMEM((_K,), jnp.int32),
            pltpu.VMEM((_K,), jnp.int32),
            pltpu.VMEM((_K, H), _F32),
            pltpu.VMEM((_K, H), _F32),
            pltpu.VMEM_SHARED((n_segs, H), _F32),
            pltpu.SemaphoreType.DMA,
            pltpu.SemaphoreType.DMA,
            pltpu.VMEM((per_w + 32,), jnp.int32),
            pltpu.VMEM((per_w // _K, _K), _F32),
            pltpu.VMEM((seg_rows, _K), _F32),
            pltpu.VMEM((seg_rows, _K), _F32),
            pltpu.VMEM((seg_rows,), jnp.int32),
            pltpu.VMEM_SHARED((seg_rows, _K), _F32),
        ]
    else:
        scratch = [
            pltpu.VMEM((n_chunks, _K), jnp.int32),
            pltpu.VMEM((n_chunks, _K), jnp.int32),
            pltpu.VMEM((_K, H), _F32),
            pltpu.VMEM((_K, H), _F32),
            pltpu.VMEM_SHARED((n_segs, H), _F32),
            pltpu.SemaphoreType.DMA,
            pltpu.SemaphoreType.DMA,
        ]
    fn = pl.kernel(
        body,
        out_type=out_type,
        mesh=mesh,
        compiler_params=pltpu.CompilerParams(needs_layout_passes=False),
        scratch_types=scratch,
        name=f"row_seg_accum_{mode}" + (f"_den{item_off}" if with_den else ""),
    )
    if with_den:
        return fn(table, ids_flat, den_vals)
    return fn(table, src3, dst3)


def _scalar_seg_sum(ids2, vals2, n_items, n_segs):
    per_w = n_items // _NW
    n_groups = per_w // 16
    seg_rows = n_segs // _K
    mesh = plsc.VectorSubcoreMesh(core_axis_name="c", subcore_axis_name="s",
                                  num_cores=_NC, num_subcores=_NS)
    have_vals = vals2 is not None

    def body(*refs):
        if have_vals:
            (ids_ref, vals_ref, out_ref, ids_buf, vals_buf, s2d, e2d,
             idx_vm, zrow, acc) = refs
        else:
            (ids_ref, out_ref, ids_buf, vals_buf, s2d, e2d,
             idx_vm, zrow, acc) = refs
        c = lax.axis_index("c")
        s = lax.axis_index("s")
        wid = c * _NS + s

        neg1 = jnp.full((16,), -1, jnp.int32)
        ids_buf[pl.ds(0, 16)] = neg1
        ids_buf[pl.ds(16 + per_w, 16)] = neg1
        off = pl.multiple_of(wid * per_w, _K)
        pltpu.sync_copy(ids_ref.at[pl.ds(off, per_w)],
                        ids_buf.at[pl.ds(16, per_w)])
        if have_vals:
            pltpu.sync_copy(vals_ref.at[pl.ds(off, per_w)], vals_buf)

        def zero2d(r, carry):
            for j in range(_K // 16):
                s2d[r, pl.ds(j * 16, 16)] = jnp.zeros((16,), _F32)
                e2d[r, pl.ds(j * 16, 16)] = jnp.zeros((16,), _F32)
            return carry
        lax.fori_loop(0, seg_rows, zero2d, 0)
        for j in range(_K // 16):
            zrow[pl.ds(j * 16, 16)] = jnp.zeros((16,), _F32)
        idx_vm[pl.ds(0, 16)] = lax.iota(jnp.int32, 16)
        idx_vm[pl.ds(16, 16)] = lax.iota(jnp.int32, 16) + 16

        @pl.when(s == 0)
        def _():
            for r in range(seg_rows):
                pltpu.sync_copy(zrow, acc.at[r])
        plsc.subcore_barrier()

        ones16 = jnp.ones((16,), _F32)

        def group(g, carry):
            base = g * 16
            ids = ids_buf[pl.ds(16 + base, 16)]
            prv = ids_buf[pl.ds(15 + base, 16)]
            nxt = ids_buf[pl.ds(17 + base, 16)]
            v = vals_buf[pl.ds(base, 16)] if have_vals else ones16
            pc = plsc.cumsum(v) + carry
            hi = lax.shift_right_arithmetic(ids, 7)
            lo = lax.bitwise_and(ids, 127)
            plsc.store_scatter(s2d, [hi, lo], pc - v, mask=ids != prv)
            plsc.store_scatter(e2d, [hi, lo], pc, mask=ids != nxt)
            return pc[15]
        lax.fori_loop(0, n_groups, group, jnp.float32(0.0))

        def diff(r, carry):
            for j in range(_K // 16):
                sl = pl.ds(j * 16, 16)
                e2d[r, sl] = e2d[r, sl] - s2d[r, sl]
            return carry
        lax.fori_loop(0, seg_rows, diff, 0)
        pltpu.sync_copy(e2d, acc.at[idx_vm], add=True)
        plsc.subcore_barrier()

        @pl.when(s == 0)
        def _():
            pltpu.sync_copy(acc, out_ref.at[c])

    scratch = [
        pltpu.VMEM((per_w + 32,), jnp.int32),
        pltpu.VMEM((per_w,), _F32),
        pltpu.VMEM((seg_rows, _K), _F32),
        pltpu.VMEM((seg_rows, _K), _F32),
        pltpu.VMEM((seg_rows,), jnp.int32),
        pltpu.VMEM((_K,), _F32),
        pltpu.VMEM_SHARED((seg_rows, _K), _F32),
    ]
    fn = pl.kernel(
        body,
        out_type=jax.ShapeDtypeStruct((_NC, seg_rows, _K), _F32),
        mesh=mesh,
        compiler_params=pltpu.CompilerParams(needs_layout_passes=False),
        scratch_types=scratch,
        name="scalar_seg_sum" + ("_vals" if have_vals else "_ones"),
    )
    return fn(ids2, vals2) if have_vals else fn(ids2)


def _encode(xt, W, b, br):
    d, n = xt.shape

    def body(xt_ref, w_ref, b_ref, o_ref):
        o_ref[...] = jnp.maximum(
            lax.dot_general(xt_ref[...], w_ref[...],
                            (((0,), (0,)), ((), ())),
                            preferred_element_type=_F32)
            + b_ref[...], 0.0)

    return pl.pallas_call(
        body,
        grid=(n // br,),
        in_specs=[pl.BlockSpec((d, br), lambda i: (0, i)),
                  pl.BlockSpec((d, H), lambda i: (0, 0)),
                  pl.BlockSpec((1, H), lambda i: (0, 0))],
        out_specs=pl.BlockSpec((br, H), lambda i: (i, 0)),
        out_shape=jax.ShapeDtypeStruct((n, H), _F32),
    )(xt, W, b)


def _self_update(hb, W, b, br):
    n = hb.shape[0]

    def body(x_ref, w_ref, b_ref, o_ref):
        x = x_ref[...]
        o_ref[...] = x + jnp.maximum(
            jnp.dot(x, w_ref[...], preferred_element_type=_F32) + b_ref[...],
            0.0)

    return pl.pallas_call(
        body,
        grid=(n // br,),
        in_specs=[pl.BlockSpec((br, H), lambda i: (i, 0)),
                  pl.BlockSpec((H, H), lambda i: (0, 0)),
                  pl.BlockSpec((1, H), lambda i: (0, 0))],
        out_specs=pl.BlockSpec((br, H), lambda i: (i, 0)),
        out_shape=jax.ShapeDtypeStruct((n, H), _F32),
    )(hb, W, b)


def _upd2_score(hb, W2, b2, Wa1, ba1, Wa2, ba2, br, row0, nrows):
    n = nrows
    blk0 = row0 // br

    def body(hb_ref, w2u_ref, b2u_ref, w1_ref, b1_ref, w2_ref, b2_ref,
             y_ref, ex_ref):
        h2 = hb_ref[...]
        h3 = h2 + jnp.maximum(
            jnp.dot(h2, w2u_ref[...], preferred_element_type=_F32)
            + b2u_ref[...], 0.0)
        t = jnp.tanh(jnp.dot(h3, w1_ref[...], preferred_element_type=_F32)
                     + b1_ref[...])
        sc = jnp.dot(t, w2_ref[...], preferred_element_type=_F32) + b2_ref[...]
        ex = jnp.exp(sc)
        y_ref[...] = h3 * ex
        ex_ref[...] = jnp.reshape(ex, (br // _K, _K))

    return pl.pallas_call(
        body,
        grid=(n // br,),
        in_specs=[pl.BlockSpec((br, H), lambda i: (i + blk0, 0)),
                  pl.BlockSpec((H, H), lambda i: (0, 0)),
                  pl.BlockSpec((1, H), lambda i: (0, 0)),
                  pl.BlockSpec((H, H // 2), lambda i: (0, 0)),
                  pl.BlockSpec((1, H // 2), lambda i: (0, 0)),
                  pl.BlockSpec((H // 2, 1), lambda i: (0, 0)),
                  pl.BlockSpec((1, 1), lambda i: (0, 0))],
        out_specs=[pl.BlockSpec((br, H), lambda i: (i, 0)),
                   pl.BlockSpec((br // _K, _K), lambda i: (i, 0))],
        out_shape=[jax.ShapeDtypeStruct((n, H), _F32),
                   jax.ShapeDtypeStruct((n // _K, _K), _F32)],
    )(hb, W2, b2, Wa1, ba1, Wa2, ba2)


def _ln_blk(x, g, b):
    m = jnp.mean(x, axis=-1, keepdims=True)
    v = jnp.mean((x - m) ** 2, axis=-1, keepdims=True)
    return (x - m) / jnp.sqrt(v + 1e-5) * g + b


def _head(xq, aggs, degp, poolp, denp, Wq, bq, W_upd, b_upd, ln_g, ln_b,
          Wc, bc, ln_gc, ln_bc, Wbo1, bbo1, Wbo2, bbo2,
          Wwk1, bwk1, Wwk2, bwk2, br):
    dq = xq.shape[1]

    def body(xq_ref, a0_ref, a1_ref, a2_ref, dg_ref,
             ppa_ref, ppb_ref, dna_ref, dnb_ref,
             wq_ref, bq_ref, wu_ref, bu_ref, lg_ref, lb_ref,
             wc_ref, bc_ref, lgc_ref, lbc_ref,
             w1b_ref, b1b_ref, w2b_ref, b2b_ref,
             w1k_ref, b1k_ref, w2k_ref, b2k_ref,
             ob_ref, ow_ref):
        hq = jnp.maximum(
            jnp.dot(xq_ref[...], wq_ref[...], preferred_element_type=_F32)
            + bq_ref[...], 0.0)
        dg = dg_ref[...]
        deg = jnp.maximum(dg[0] + dg[1], 1.0)
        wu = wu_ref[...]
        bu = bu_ref[...]
        lg = lg_ref[...]
        lb = lb_ref[...]
        for l, a_ref in enumerate((a0_ref, a1_ref, a2_ref)):
            a = a_ref[...]
            agg = (a[0] + a[1]) / deg
            u = jnp.maximum(
                jnp.dot(agg, wu[l], preferred_element_type=_F32) + bu[l], 0.0)
            hq = _ln_blk(hq + u, lg[l], lb[l])
        ppa = ppa_ref[...]
        ppb = ppb_ref[...]
        dna = dna_ref[...]
        dnb = dnb_ref[...]
        den = dna[0] + dna[1] + dnb[0] + dnb[1]
        pooled = (ppa[0] + ppa[1] + ppb[0] + ppb[1]) / (den + 1e-16)
        wc = wc_ref[...]
        comb = (jnp.dot(hq, wc[0:H], preferred_element_type=_F32)
                + jnp.dot(pooled, wc[H:2 * H], preferred_element_type=_F32)
                + bc_ref[...])
        r = jax.nn.gelu(_ln_blk(comb, lgc_ref[...], lbc_ref[...]))
        hb1 = jnp.maximum(
            jnp.dot(r, w1b_ref[...], preferred_element_type=_F32)
            + b1b_ref[...], 0.0)
        ob_ref[...] = (jnp.dot(hb1, w2b_ref[...], preferred_element_type=_F32)
                       + b2b_ref[...])
        hk1 = jnp.maximum(
            jnp.dot(r, w1k_ref[...], preferred_element_type=_F32)
            + b1k_ref[...], 0.0)
        ow_ref[...] = (jnp.dot(hk1, w2k_ref[...], preferred_element_type=_F32)
                       + b2k_ref[...])

    full = lambda *shape: pl.BlockSpec(shape, lambda i: (0,) * len(shape))
    return pl.pallas_call(
        body,
        grid=(B // br,),
        in_specs=[
            pl.BlockSpec((br, dq), lambda i: (i, 0)),
            pl.BlockSpec((2, br, H), lambda i: (0, i, 0)),
            pl.BlockSpec((2, br, H), lambda i: (0, i, 0)),
            pl.BlockSpec((2, br, H), lambda i: (0, i, 0)),
            pl.BlockSpec((2, br, 1), lambda i: (0, i, 0)),
            pl.BlockSpec((2, br, H), lambda i: (0, i, 0)),
            pl.BlockSpec((2, br, H), lambda i: (0, i, 0)),
            pl.BlockSpec((2, br, 1), lambda i: (0, i, 0)),
            pl.BlockSpec((2, br, 1), lambda i: (0, i, 0)),
            full(dq, H), full(1, H),
            full(L, H, H), full(L, H), full(L, H), full(L, H),
            full(2 * H, H), full(1, H), full(1, H), full(1, H),
            full(H, H // 2), full(1, H // 2), full(H // 2, 1), full(1, 1),
            full(H, H // 2), full(1, H // 2), full(H // 2, 1), full(1, 1),
        ],
        out_specs=[pl.BlockSpec((br, 1), lambda i: (i, 0)),
                   pl.BlockSpec((br, 1), lambda i: (i, 0))],
        out_shape=[jax.ShapeDtypeStruct((B, 1), _F32),
                   jax.ShapeDtypeStruct((B, 1), _F32)],
    )(xq, aggs[0], aggs[1], aggs[2], degp, poolp[0], poolp[1],
      denp[0], denp[1], Wq, bq, W_upd, b_upd,
      ln_g, ln_b, Wc, bc, ln_gc, ln_bc, Wbo1, bbo1, Wbo2, bbo2,
      Wwk1, bwk1, Wwk2, bwk2)


def kernel(x_query, x_ball, edge_src, edge_dst, ball_batch,
           Wq, bq, Wb, bb, W_upd, b_upd, W_self, b_self, ln_g, ln_b,
           Wa1, ba1, Wa2, ba2, Wc, bc, ln_gc, ln_bc,
           Wbo1, bbo1, Wbo2, bbo2, Wwk1, bwk1, Wwk2, bwk2):
    n_chunks = E // _NW // _K

    src3 = edge_src.reshape(_NW, n_chunks, _K)
    dst3 = edge_dst.reshape(_NW, n_chunks, _K)

    hb0 = _encode(x_ball.T, Wb, bb.reshape(1, H), br=4096)
    degp = _scalar_seg_sum(edge_dst, None, n_items=E, n_segs=B)
    hb0c, _ = lax.optimization_barrier((hb0, degp))
    agg0 = _row_seg_accum(hb0c, src3, dst3, n_items=E, n_segs=B, mode="gather")
    hb1 = _self_update(hb0, W_self[0], b_self[0].reshape(1, H), br=4096)
    hb1c, _ = lax.optimization_barrier((hb1, agg0))
    agg1 = _row_seg_accum(hb1c, src3, dst3, n_items=E, n_segs=B, mode="gather")
    hb2 = _self_update(hb1, W_self[1], b_self[1].reshape(1, H), br=4096)
    hb2c, _ = lax.optimization_barrier((hb2, agg1))
    agg2 = _row_seg_accum(hb2c, src3, dst3, n_items=E, n_segs=B, mode="gather")
    aggs = [agg0, agg1, agg2]

    half = NBALL // 2
    sargs = (W_self[2], b_self[2].reshape(1, H), Wa1,
             ba1.reshape(1, H // 2), Wa2, ba2.reshape(1, 1))
    ya, exa = _upd2_score(hb2, *sargs, br=2048, row0=0, nrows=half)
    yb, exb = _upd2_score(hb2, *sargs, br=2048, row0=half, nrows=half)
    yac, _ = lax.optimization_barrier((ya, agg2))
    pool_a, den_a = _row_seg_accum(yac, None, None, n_items=half, n_segs=B,
                                   mode="linear", ids_flat=ball_batch,
                                   den_vals=exa, item_off=0)
    ybc, _ = lax.optimization_barrier((yb, pool_a))
    pool_b, den_b = _row_seg_accum(ybc, None, None, n_items=half, n_segs=B,
                                   mode="linear", ids_flat=ball_batch,
                                   den_vals=exb, item_off=half)
    poolp = (pool_a, pool_b)
    denp = (den_a.reshape(_NC, B, 1), den_b.reshape(_NC, B, 1))

    xq = jnp.pad(x_query, ((0, 0), (0, 1)))
    Wqp = jnp.pad(Wq, ((0, 1), (0, 0)))
    boundary, wicket = _head(
        xq, aggs, degp.reshape(_NC, B, 1), poolp, denp,
        Wqp, bq.reshape(1, H), W_upd, b_upd,
        ln_g, ln_b, Wc, bc.reshape(1, H), ln_gc.reshape(1, H),
        ln_bc.reshape(1, H), Wbo1, bbo1.reshape(1, H // 2), Wbo2,
        bbo2.reshape(1, 1), Wwk1, bwk1.reshape(1, H // 2), Wwk2,
        bwk2.reshape(1, 1), br=1024)
    return boundary, wicket


# speedup vs baseline: 1.0168x; 1.0019x over previous
import functools

import jax
import jax.numpy as jnp
from jax import lax
from jax.experimental import pallas as pl
from jax.experimental.pallas import tpu as pltpu
from jax.experimental.pallas import tpu_sc as plsc

B = 4096
NBALL = 131072
E = 131072
H = 128
L = 3

_NC = 2
_NS = 16
_NW = _NC * _NS
_K = 128
_F32 = jnp.float32


def _row_seg_accum(table, src3, dst3, n_items, n_segs, mode,
                   ids_flat=None, den_vals=None, item_off=0):
    per_w = n_items // _NW
    n_chunks = per_w // _K
    rows_per_sub = n_segs // _NS
    seg_rows = n_segs // _K
    with_den = den_vals is not None
    mesh = plsc.VectorSubcoreMesh(core_axis_name="c", subcore_axis_name="s",
                                  num_cores=_NC, num_subcores=_NS)

    def body(*refs):
        if with_den:
            (table_ref, ids_ref, vals_ref,
             out_ref, den_ref,
             dst_v0, dst_v1, rows0, rows1, acc, sem0, sem1,
             ids_buf, vals2d, s2d, e2d, idx_vm, den_acc) = refs
            src_vm = dst_vm = None
        else:
            (table_ref, src_ref, dst_ref, out_ref,
             src_vm, dst_vm, rows0, rows1, acc, sem0, sem1) = refs
        c = lax.axis_index("c")
        s = lax.axis_index("s")
        wid = c * _NS + s

        def zero_row(i, carry):
            for j in range(H // 16):
                rows0[i, pl.ds(j * 16, 16)] = jnp.zeros((16,), _F32)
            return carry
        lax.fori_loop(0, _K, zero_row, 0)
        for t in range(rows_per_sub // _K):
            pltpu.sync_copy(rows0, acc.at[pl.ds(s * rows_per_sub + t * _K, _K)])
        if with_den:
            @pl.when(s == 0)
            def _():
                pltpu.sync_copy(rows0.at[pl.ds(0, seg_rows)], den_acc)
        plsc.subcore_barrier()

        if with_den:
            neg1 = jnp.full((16,), -1, jnp.int32)
            ids_buf[pl.ds(0, 16)] = neg1
            ids_buf[pl.ds(16 + per_w, 16)] = neg1
            off0 = pl.multiple_of(item_off + wid * per_w, _K)
            pltpu.sync_copy(ids_ref.at[pl.ds(off0, per_w)],
                            ids_buf.at[pl.ds(16, per_w)])
            pltpu.sync_copy(vals_ref.at[pl.ds(wid * (per_w // _K),
                                              per_w // _K)], vals2d)

            def zero2d(r, carry):
                for j in range(_K // 16):
                    s2d[r, pl.ds(j * 16, 16)] = jnp.zeros((16,), _F32)
                    e2d[r, pl.ds(j * 16, 16)] = jnp.zeros((16,), _F32)
                return carry
            lax.fori_loop(0, seg_rows, zero2d, 0)
            idx_vm[pl.ds(0, 16)] = lax.iota(jnp.int32, 16)
            idx_vm[pl.ds(16, 16)] = lax.iota(jnp.int32, 16) + 16
        else:
            pltpu.sync_copy(src_ref.at[wid], src_vm)
            pltpu.sync_copy(dst_ref.at[wid], dst_vm)

        def fetch(i, buf, sem):
            if mode == "gather":
                return pltpu.async_copy(table_ref.at[src_vm.at[i]], buf, sem)
            off = pl.multiple_of(wid * per_w, _K) + i * _K
            return pltpu.async_copy(table_ref.at[pl.ds(off, _K)], buf, sem)

        def wait(i, buf, sem):
            if mode == "gather":
                pltpu.make_async_copy(table_ref.at[src_vm.at[i]], buf, sem).wait()
            else:
                off = pl.multiple_of(wid * per_w, _K) + i * _K
                pltpu.make_async_copy(table_ref.at[pl.ds(off, _K)], buf, sem).wait()

        def dst_idx(i, which):
            if not with_den:
                return dst_vm.at[i]
            buf = dst_v0 if which == 0 else dst_v1
            off = pl.multiple_of(item_off + wid * per_w, _K) + i * _K
            pltpu.sync_copy(ids_ref.at[pl.ds(off, _K)], buf)
            return buf

        fetch(0, rows0, sem0)

        def scan_groups(i, carry):
            if not with_den:
                return carry
            for gj in range(8):
                base = i * _K + gj * 16
                ids = ids_buf[pl.ds(16 + base, 16)]
                prv = ids_buf[pl.ds(15 + base, 16)]
                nxt = ids_buf[pl.ds(17 + base, 16)]
                v = vals2d[i, pl.ds(gj * 16, 16)]
                pc = plsc.cumsum(v) + carry
                hi = lax.shift_right_arithmetic(ids, 7)
                lo = lax.bitwise_and(ids, 127)
                plsc.store_scatter(s2d, [hi, lo], pc - v, mask=ids != prv)
                plsc.store_scatter(e2d, [hi, lo], pc, mask=ids != nxt)
                carry = pc[15]
            return carry

        def chunk2(k, carry):
            i0 = 2 * k
            i1 = 2 * k + 1
            wait(i0, rows0, sem0)
            fetch(i1, rows1, sem1)
            pltpu.sync_copy(rows0, acc.at[dst_idx(i0, 0)], add=True)
            carry = scan_groups(i0, carry)
            wait(i1, rows1, sem1)

            @pl.when(i0 + 2 < n_chunks)
            def _():
                fetch(i0 + 2, rows0, sem0)
            pltpu.sync_copy(rows1, acc.at[dst_idx(i1, 1)], add=True)
            carry = scan_groups(i1, carry)
            return carry
        lax.fori_loop(0, n_chunks // 2, chunk2, jnp.float32(0.0))

        if with_den:
            def diff(r, carry):
                for j in range(_K // 16):
                    sl = pl.ds(j * 16, 16)
                    e2d[r, sl] = e2d[r, sl] - s2d[r, sl]
                return carry
            lax.fori_loop(0, seg_rows, diff, 0)
            pltpu.sync_copy(e2d, den_acc.at[idx_vm], add=True)

        plsc.subcore_barrier()
        for t in range(rows_per_sub // _K):
            r0 = s * rows_per_sub + t * _K
            pltpu.sync_copy(acc.at[pl.ds(r0, _K)], out_ref.at[c, pl.ds(r0, _K)])
        if with_den:
            @pl.when(s == 0)
            def _():
                pltpu.sync_copy(den_acc, den_ref.at[c])

    out_type = jax.ShapeDtypeStruct((_NC, n_segs, H), _F32)
    if with_den:
        out_type = [out_type,
                    jax.ShapeDtypeStruct((_NC, seg_rows, _K), _F32)]
        scratch = [
            pltpu.VMEM((_K,), jnp.int32),
            pltpu.VMEM((_K,), jnp.int32),
            pltpu.VMEM((_K, H), _F32),
            pltpu.VMEM((_K, H), _F32),
            pltpu.VMEM_SHARED((n_segs, H), _F32),
            pltpu.SemaphoreType.DMA,
            pltpu.SemaphoreType.DMA,
            pltpu.VMEM((per_w + 32,), jnp.int32),
            pltpu.VMEM((per_w // _K, _K), _F32),
            pltpu.VMEM((seg_rows, _K), _F32),
            pltpu.VMEM((seg_rows, _K), _F32),
            pltpu.VMEM((seg_rows,), jnp.int32),
            pltpu.VMEM_SHARED((seg_rows, _K), _F32),
        ]
    else:
        scratch = [
            pltpu.VMEM((n_chunks, _K), jnp.int32),
            pltpu.VMEM((n_chunks, _K), jnp.int32),
            pltpu.VMEM((_K, H), _F32),
            pltpu.VMEM((_K, H), _F32),
            pltpu.VMEM_SHARED((n_segs, H), _F32),
            pltpu.SemaphoreType.DMA,
            pltpu.SemaphoreType.DMA,
        ]
    fn = pl.kernel(
        body,
        out_type=out_type,
        mesh=mesh,
        compiler_params=pltpu.CompilerParams(needs_layout_passes=False),
        scratch_types=scratch,
        name=f"row_seg_accum_{mode}" + (f"_den{item_off}" if with_den else ""),
    )
    if with_den:
        return fn(table, ids_flat, den_vals)
    return fn(table, src3, dst3)


def _scalar_seg_sum(ids2, vals2, n_items, n_segs):
    per_w = n_items // _NW
    n_groups = per_w // 16
    seg_rows = n_segs // _K
    mesh = plsc.VectorSubcoreMesh(core_axis_name="c", subcore_axis_name="s",
                                  num_cores=_NC, num_subcores=_NS)
    have_vals = vals2 is not None

    def body(*refs):
        if have_vals:
            (ids_ref, vals_ref, out_ref, ids_buf, vals_buf, s2d, e2d,
             idx_vm, zrow, acc) = refs
        else:
            (ids_ref, out_ref, ids_buf, vals_buf, s2d, e2d,
             idx_vm, zrow, acc) = refs
        c = lax.axis_index("c")
        s = lax.axis_index("s")
        wid = c * _NS + s

        neg1 = jnp.full((16,), -1, jnp.int32)
        ids_buf[pl.ds(0, 16)] = neg1
        ids_buf[pl.ds(16 + per_w, 16)] = neg1
        off = pl.multiple_of(wid * per_w, _K)
        pltpu.sync_copy(ids_ref.at[pl.ds(off, per_w)],
                        ids_buf.at[pl.ds(16, per_w)])
        if have_vals:
            pltpu.sync_copy(vals_ref.at[pl.ds(off, per_w)], vals_buf)

        def zero2d(r, carry):
            for j in range(_K // 16):
                s2d[r, pl.ds(j * 16, 16)] = jnp.zeros((16,), _F32)
                e2d[r, pl.ds(j * 16, 16)] = jnp.zeros((16,), _F32)
            return carry
        lax.fori_loop(0, seg_rows, zero2d, 0)
        for j in range(_K // 16):
            zrow[pl.ds(j * 16, 16)] = jnp.zeros((16,), _F32)
        idx_vm[pl.ds(0, 16)] = lax.iota(jnp.int32, 16)
        idx_vm[pl.ds(16, 16)] = lax.iota(jnp.int32, 16) + 16

        @pl.when(s == 0)
        def _():
            for r in range(seg_rows):
                pltpu.sync_copy(zrow, acc.at[r])
        plsc.subcore_barrier()

        ones16 = jnp.ones((16,), _F32)

        def group(g, carry):
            base = g * 16
            ids = ids_buf[pl.ds(16 + base, 16)]
            prv = ids_buf[pl.ds(15 + base, 16)]
            nxt = ids_buf[pl.ds(17 + base, 16)]
            v = vals_buf[pl.ds(base, 16)] if have_vals else ones16
            pc = plsc.cumsum(v) + carry
            hi = lax.shift_right_arithmetic(ids, 7)
            lo = lax.bitwise_and(ids, 127)
            plsc.store_scatter(s2d, [hi, lo], pc - v, mask=ids != prv)
            plsc.store_scatter(e2d, [hi, lo], pc, mask=ids != nxt)
            return pc[15]
        lax.fori_loop(0, n_groups, group, jnp.float32(0.0))

        def diff(r, carry):
            for j in range(_K // 16):
                sl = pl.ds(j * 16, 16)
                e2d[r, sl] = e2d[r, sl] - s2d[r, sl]
            return carry
        lax.fori_loop(0, seg_rows, diff, 0)
        pltpu.sync_copy(e2d, acc.at[idx_vm], add=True)
        plsc.subcore_barrier()

        @pl.when(s == 0)
        def _():
            pltpu.sync_copy(acc, out_ref.at[c])

    scratch = [
        pltpu.VMEM((per_w + 32,), jnp.int32),
        pltpu.VMEM((per_w,), _F32),
        pltpu.VMEM((seg_rows, _K), _F32),
        pltpu.VMEM((seg_rows, _K), _F32),
        pltpu.VMEM((seg_rows,), jnp.int32),
        pltpu.VMEM((_K,), _F32),
        pltpu.VMEM_SHARED((seg_rows, _K), _F32),
    ]
    fn = pl.kernel(
        body,
        out_type=jax.ShapeDtypeStruct((_NC, seg_rows, _K), _F32),
        mesh=mesh,
        compiler_params=pltpu.CompilerParams(needs_layout_passes=False),
        scratch_types=scratch,
        name="scalar_seg_sum" + ("_vals" if have_vals else "_ones"),
    )
    return fn(ids2, vals2) if have_vals else fn(ids2)


def _encode(xt, W, b, br):
    d, n = xt.shape

    def body(xt_ref, w_ref, b_ref, o_ref):
        o_ref[...] = jnp.maximum(
            lax.dot_general(xt_ref[...], w_ref[...],
                            (((0,), (0,)), ((), ())),
                            preferred_element_type=_F32)
            + b_ref[...], 0.0)

    return pl.pallas_call(
        body,
        grid=(n // br,),
        in_specs=[pl.BlockSpec((d, br), lambda i: (0, i)),
                  pl.BlockSpec((d, H), lambda i: (0, 0)),
                  pl.BlockSpec((1, H), lambda i: (0, 0))],
        out_specs=pl.BlockSpec((br, H), lambda i: (i, 0)),
        out_shape=jax.ShapeDtypeStruct((n, H), _F32),
    )(xt, W, b)


def _self_update(hb, W, b, br):
    n = hb.shape[0]

    def body(x_ref, w_ref, b_ref, o_ref):
        x = x_ref[...]
        o_ref[...] = x + jnp.maximum(
            jnp.dot(x, w_ref[...], preferred_element_type=_F32) + b_ref[...],
            0.0)

    return pl.pallas_call(
        body,
        grid=(n // br,),
        in_specs=[pl.BlockSpec((br, H), lambda i: (i, 0)),
                  pl.BlockSpec((H, H), lambda i: (0, 0)),
                  pl.BlockSpec((1, H), lambda i: (0, 0))],
        out_specs=pl.BlockSpec((br, H), lambda i: (i, 0)),
        out_shape=jax.ShapeDtypeStruct((n, H), _F32),
    )(hb, W, b)


def _upd2_score(hb, W2, b2, Wa1, ba1, Wa2, ba2, br, row0, nrows):
    n = nrows
    blk0 = row0 // br

    def body(hb_ref, w2u_ref, b2u_ref, w1_ref, b1_ref, w2_ref, b2_ref,
             y_ref, ex_ref):
        h2 = hb_ref[...]
        h3 = h2 + jnp.maximum(
            jnp.dot(h2, w2u_ref[...], preferred_element_type=_F32)
            + b2u_ref[...], 0.0)
        t = jnp.tanh(jnp.dot(h3, w1_ref[...], preferred_element_type=_F32)
                     + b1_ref[...])
        sc = jnp.dot(t, w2_ref[...], preferred_element_type=_F32) + b2_ref[...]
        ex = jnp.exp(sc)
        y_ref[...] = h3 * ex
        ex_ref[...] = jnp.reshape(ex, (br // _K, _K))

    return pl.pallas_call(
        body,
        grid=(n // br,),
        in_specs=[pl.BlockSpec((br, H), lambda i: (i + blk0, 0)),
                  pl.BlockSpec((H, H), lambda i: (0, 0)),
                  pl.BlockSpec((1, H), lambda i: (0, 0)),
                  pl.BlockSpec((H, H // 2), lambda i: (0, 0)),
                  pl.BlockSpec((1, H // 2), lambda i: (0, 0)),
                  pl.BlockSpec((H // 2, 1), lambda i: (0, 0)),
                  pl.BlockSpec((1, 1), lambda i: (0, 0))],
        out_specs=[pl.BlockSpec((br, H), lambda i: (i, 0)),
                   pl.BlockSpec((br // _K, _K), lambda i: (i, 0))],
        out_shape=[jax.ShapeDtypeStruct((n, H), _F32),
                   jax.ShapeDtypeStruct((n // _K, _K), _F32)],
    )(hb, W2, b2, Wa1, ba1, Wa2, ba2)


def _ln_blk(x, g, b):
    m = jnp.mean(x, axis=-1, keepdims=True)
    v = jnp.mean((x - m) ** 2, axis=-1, keepdims=True)
    return (x - m) / jnp.sqrt(v + 1e-5) * g + b


def _head(xq, aggs, degp, poolp, denp, Wq, bq, W_upd, b_upd, ln_g, ln_b,
          Wc, bc, ln_gc, ln_bc, Wbo1, bbo1, Wbo2, bbo2,
          Wwk1, bwk1, Wwk2, bwk2, br):
    dq = xq.shape[1]

    def body(xq_ref, a0_ref, a1_ref, a2_ref, dg_ref,
             ppa_ref, ppb_ref, dna_ref, dnb_ref,
             wq_ref, bq_ref, wu_ref, bu_ref, lg_ref, lb_ref,
             wc_ref, bc_ref, lgc_ref, lbc_ref,
             w1b_ref, b1b_ref, w2b_ref, b2b_ref,
             w1k_ref, b1k_ref, w2k_ref, b2k_ref,
             ob_ref, ow_ref):
        hq = jnp.maximum(
            jnp.dot(xq_ref[...], wq_ref[...], preferred_element_type=_F32)
            + bq_ref[...], 0.0)
        dg = dg_ref[...]
        deg = jnp.maximum(dg[0] + dg[1], 1.0)
        wu = wu_ref[...]
        bu = bu_ref[...]
        lg = lg_ref[...]
        lb = lb_ref[...]
        for l, a_ref in enumerate((a0_ref, a1_ref, a2_ref)):
            a = a_ref[...]
            agg = (a[0] + a[1]) / deg
            u = jnp.maximum(
                jnp.dot(agg, wu[l], preferred_element_type=_F32) + bu[l], 0.0)
            hq = _ln_blk(hq + u, lg[l], lb[l])
        ppa = ppa_ref[...]
        ppb = ppb_ref[...]
        dna = dna_ref[...]
        dnb = dnb_ref[...]
        den = dna[0] + dna[1] + dnb[0] + dnb[1]
        pooled = (ppa[0] + ppa[1] + ppb[0] + ppb[1]) / (den + 1e-16)
        wc = wc_ref[...]
        comb = (jnp.dot(hq, wc[0:H], preferred_element_type=_F32)
                + jnp.dot(pooled, wc[H:2 * H], preferred_element_type=_F32)
                + bc_ref[...])
        r = jax.nn.gelu(_ln_blk(comb, lgc_ref[...], lbc_ref[...]))
        hb1 = jnp.maximum(
            jnp.dot(r, w1b_ref[...], preferred_element_type=_F32)
            + b1b_ref[...], 0.0)
        ob_ref[...] = (jnp.dot(hb1, w2b_ref[...], preferred_element_type=_F32)
                       + b2b_ref[...])
        hk1 = jnp.maximum(
            jnp.dot(r, w1k_ref[...], preferred_element_type=_F32)
            + b1k_ref[...], 0.0)
        ow_ref[...] = (jnp.dot(hk1, w2k_ref[...], preferred_element_type=_F32)
                       + b2k_ref[...])

    full = lambda *shape: pl.BlockSpec(shape, lambda i: (0,) * len(shape))
    return pl.pallas_call(
        body,
        grid=(B // br,),
        in_specs=[
            pl.BlockSpec((br, dq), lambda i: (i, 0)),
            pl.BlockSpec((2, br, H), lambda i: (0, i, 0)),
            pl.BlockSpec((2, br, H), lambda i: (0, i, 0)),
            pl.BlockSpec((2, br, H), lambda i: (0, i, 0)),
            pl.BlockSpec((2, br, 1), lambda i: (0, i, 0)),
            pl.BlockSpec((2, br, H), lambda i: (0, i, 0)),
            pl.BlockSpec((2, br, H), lambda i: (0, i, 0)),
            pl.BlockSpec((2, br, 1), lambda i: (0, i, 0)),
            pl.BlockSpec((2, br, 1), lambda i: (0, i, 0)),
            full(dq, H), full(1, H),
            full(L, H, H), full(L, H), full(L, H), full(L, H),
            full(2 * H, H), full(1, H), full(1, H), full(1, H),
            full(H, H // 2), full(1, H // 2), full(H // 2, 1), full(1, 1),
            full(H, H // 2), full(1, H // 2), full(H // 2, 1), full(1, 1),
        ],
        out_specs=[pl.BlockSpec((br, 1), lambda i: (i, 0)),
                   pl.BlockSpec((br, 1), lambda i: (i, 0))],
        out_shape=[jax.ShapeDtypeStruct((B, 1), _F32),
                   jax.ShapeDtypeStruct((B, 1), _F32)],
    )(xq, aggs[0], aggs[1], aggs[2], degp, poolp[0], poolp[1],
      denp[0], denp[1], Wq, bq, W_upd, b_upd,
      ln_g, ln_b, Wc, bc, ln_gc, ln_bc, Wbo1, bbo1, Wbo2, bbo2,
      Wwk1, bwk1, Wwk2, bwk2)


def kernel(x_query, x_ball, edge_src, edge_dst, ball_batch,
           Wq, bq, Wb, bb, W_upd, b_upd, W_self, b_self, ln_g, ln_b,
           Wa1, ba1, Wa2, ba2, Wc, bc, ln_gc, ln_bc,
           Wbo1, bbo1, Wbo2, bbo2, Wwk1, bwk1, Wwk2, bwk2):
    n_chunks = E // _NW // _K

    src3 = edge_src.reshape(_NW, n_chunks, _K)
    dst3 = edge_dst.reshape(_NW, n_chunks, _K)

    hb0 = _encode(x_ball.T, Wb, bb.reshape(1, H), br=4096)
    degp = _scalar_seg_sum(edge_dst, None, n_items=E, n_segs=B)
    hb0c, _ = lax.optimization_barrier((hb0, degp))
    agg0 = _row_seg_accum(hb0c, src3, dst3, n_items=E, n_segs=B, mode="gather")
    hb1 = _self_update(hb0, W_self[0], b_self[0].reshape(1, H), br=4096)
    hb1c, _ = lax.optimization_barrier((hb1, agg0))
    agg1 = _row_seg_accum(hb1c, src3, dst3, n_items=E, n_segs=B, mode="gather")
    hb2 = _self_update(hb1, W_self[1], b_self[1].reshape(1, H), br=4096)
    hb2c, _ = lax.optimization_barrier((hb2, agg1))
    agg2 = _row_seg_accum(hb2c, src3, dst3, n_items=E, n_segs=B, mode="gather")
    aggs = [agg0, agg1, agg2]

    half = NBALL // 2
    sargs = (W_self[2], b_self[2].reshape(1, H), Wa1,
             ba1.reshape(1, H // 2), Wa2, ba2.reshape(1, 1))
    ya, exa = _upd2_score(hb2, *sargs, br=2048, row0=0, nrows=half)
    yb, exb = _upd2_score(hb2, *sargs, br=2048, row0=half, nrows=half)
    yac, _ = lax.optimization_barrier((ya, agg2))
    pool_a, den_a = _row_seg_accum(yac, None, None, n_items=half, n_segs=B,
                                   mode="linear", ids_flat=ball_batch,
                                   den_vals=exa, item_off=0)
    ybc, _ = lax.optimization_barrier((yb, pool_a))
    pool_b, den_b = _row_seg_accum(ybc, None, None, n_items=half, n_segs=B,
                                   mode="linear", ids_flat=ball_batch,
                                   den_vals=exb, item_off=half)
    poolp = (pool_a, pool_b)
    denp = (den_a.reshape(_NC, B, 1), den_b.reshape(_NC, B, 1))

    xq = jnp.pad(x_query, ((0, 0), (0, 1)))
    Wqp = jnp.pad(Wq, ((0, 1), (0, 0)))
    boundary, wicket = _head(
        xq, aggs, degp.reshape(_NC, B, 1), poolp, denp,
        Wqp, bq.reshape(1, H), W_upd, b_upd,
        ln_g, ln_b, Wc, bc.reshape(1, H), ln_gc.reshape(1, H),
        ln_bc.reshape(1, H), Wbo1, bbo1.reshape(1, H // 2), Wbo2,
        bbo2.reshape(1, 1), Wwk1, bwk1.reshape(1, H // 2), Wwk2,
        bwk2.reshape(1, 1), br=1024)
    return boundary, wicket
